# baseline (speedup 1.0000x reference)
import functools

import jax
import jax.numpy as jnp
from jax import lax
from jax.experimental import pallas as pl
from jax.experimental.pallas import tpu as pltpu
from jax.experimental.pallas import tpu_sc as plsc

_N = 32768
_D = 64
_K = 1024
_BLK = 8192
_COMMIT = 0.25
_NW = 32
_BPW = _N // _NW
_CHUNK = 128


def _tc_body(z_ref, wsq_ref, w2_ref, idx_ref, loss_ref):
    z = z_ref[...]
    zsq = jnp.sum(z * z, axis=1, keepdims=True)
    nchunk = _K // 128
    acc = None
    for j in range(nchunk):
        mm2_j = lax.dot_general(z, w2_ref[j * 128:(j + 1) * 128, :],
                                (((1,), (1,)), ((), ())),
                                preferred_element_type=jnp.float32)
        d = (zsq + wsq_ref[:, j * 128:(j + 1) * 128]) + mm2_j
        if acc is None:
            acc = d
            jb = jnp.zeros(d.shape, jnp.int32)
        else:
            jb = jnp.where(d < acc, j, jb)
            acc = jnp.minimum(acc, d)
    minval = jnp.min(acc, axis=1, keepdims=True)
    kc = jb * 128 + lax.broadcasted_iota(jnp.int32, acc.shape, 1)
    idx = jnp.min(jnp.where(acc == minval, kc, _K),
                  axis=1, keepdims=True)
    ii = (lax.broadcasted_iota(jnp.int32, (128, 128), 0)
          == lax.broadcasted_iota(jnp.int32, (128, 128), 1)).astype(jnp.float32)
    lof = (idx & 127).astype(jnp.float32)
    hif = (idx >> 7).astype(jnp.float32)
    rows = []
    for c in range(idx.shape[0] // 128):
        cs = slice(c * 128, (c + 1) * 128)
        lo = lax.dot_general(lof[cs, :], ii, (((0,), (0,)), ((), ())),
                             preferred_element_type=jnp.float32)
        hi = lax.dot_general(hif[cs, :], ii, (((0,), (0,)), ((), ())),
                             preferred_element_type=jnp.float32)
        rows.append((hi.astype(jnp.int32) << 7) | lo.astype(jnp.int32))
    idx_ref[...] = jnp.concatenate(rows, axis=0)
    s = jnp.reshape(jnp.sum(minval) * ((1.0 + _COMMIT) / (_N * _D)), (1, 1))
    i = pl.program_id(0)

    @pl.when(i == 0)
    def _():
        loss_ref[...] = s

    @pl.when(i != 0)
    def _():
        loss_ref[...] = loss_ref[...] + s


def _tc_call(z, wsq, W):
    grid = _N // _BLK
    return pl.pallas_call(
        _tc_body,
        grid=(grid,),
        in_specs=[
            pl.BlockSpec((_BLK, _D), lambda i: (i, 0)),
            pl.BlockSpec((1, _K), lambda i: (0, 0)),
            pl.BlockSpec((_K, _D), lambda i: (0, 0)),
        ],
        out_specs=[
            pl.BlockSpec((_BLK // 128, 128), lambda i: (i, 0)),
            pl.BlockSpec((1, 1), lambda i: (0, 0)),
        ],
        out_shape=[
            jax.ShapeDtypeStruct((_N // 128, 128), jnp.int32),
            jax.ShapeDtypeStruct((1, 1), jnp.float32),
        ],
    )(z, wsq, W)


def _sc_gather(table, idx3):
    mesh = plsc.VectorSubcoreMesh(core_axis_name="c", subcore_axis_name="s")

    @functools.partial(
        pl.kernel,
        mesh=mesh,
        out_type=jax.ShapeDtypeStruct((_N, _D), jnp.float32),
        scratch_types=[
            pltpu.VMEM((_BPW // _CHUNK, _CHUNK), jnp.int32),
            pltpu.VMEM((_BPW, _D), jnp.float32),
            pltpu.SemaphoreType.DMA,
        ],
        compiler_params=pltpu.CompilerParams(use_tc_tiling_on_sc=False),
    )
    def k(table_hbm, idx_hbm, out_hbm, idx_v, rows_v, sem):
        wid = lax.axis_index("s") * 2 + lax.axis_index("c")
        nrow = _BPW // _CHUNK
        pltpu.sync_copy(idx_hbm.at[pl.ds(wid * nrow, nrow)], idx_v)
        copies = []
        for j in range(nrow):
            copies.append(
                pltpu.async_copy(table_hbm.at[idx_v.at[j]],
                                 rows_v.at[pl.ds(j * _CHUNK, _CHUNK)], sem))
        for c in copies:
            c.wait()
        pltpu.sync_copy(rows_v, out_hbm.at[pl.ds(wid * _BPW, _BPW)])

    return k(table, idx3)


def kernel(z, W):
    wsq = jnp.sum(W ** 2, axis=1)[None, :]
    idxcc, loss2d = _tc_call(z, wsq, -2.0 * W)
    quantized = _sc_gather(W, idxcc)
    loss = loss2d[0, 0]
    return (quantized, loss, idxcc.reshape(_N, 1))


# speedup vs baseline: 1.0465x; 1.0465x over previous
import functools

import jax
import jax.numpy as jnp
from jax import lax
from jax.experimental import pallas as pl
from jax.experimental.pallas import tpu as pltpu
from jax.experimental.pallas import tpu_sc as plsc

_N = 32768
_D = 64
_K = 1024
_BLK = 8192
_COMMIT = 0.25
_NW = 32
_BPW = _N // _NW
_CHUNK = 128


def _tc_body(z_ref, wsq_ref, w2_ref, idx_ref, loss_ref):
    z = z_ref[...]
    zsq = jnp.sum(z * z, axis=1, keepdims=True)
    mm2 = lax.dot_general(z, w2_ref[...], (((1,), (1,)), ((), ())),
                          preferred_element_type=jnp.float32)
    nchunk = _K // 128
    acc = None
    for j in range(nchunk):
        d = (zsq + wsq_ref[:, j * 128:(j + 1) * 128]) + mm2[:, j * 128:(j + 1) * 128]
        if acc is None:
            acc = d
            jb = jnp.zeros(d.shape, jnp.int32)
        else:
            jb = jnp.where(d < acc, j, jb)
            acc = jnp.minimum(acc, d)
    minval = jnp.min(acc, axis=1, keepdims=True)
    kc = jb * 128 + lax.broadcasted_iota(jnp.int32, acc.shape, 1)
    idx = jnp.min(jnp.where(acc == minval, kc, _K),
                  axis=1, keepdims=True)
    idx3 = jnp.reshape(idx, (idx.shape[0] // 128, 128, 1))
    idx_ref[...] = jnp.reshape(jnp.swapaxes(idx3, 1, 2),
                               (idx.shape[0] // 128, 128))
    s = jnp.reshape(jnp.sum(minval) * ((1.0 + _COMMIT) / (_N * _D)), (1, 1))
    i = pl.program_id(0)

    @pl.when(i == 0)
    def _():
        loss_ref[...] = s

    @pl.when(i != 0)
    def _():
        loss_ref[...] = loss_ref[...] + s


def _tc_call(z, wsq, W):
    grid = _N // _BLK
    return pl.pallas_call(
        _tc_body,
        grid=(grid,),
        in_specs=[
            pl.BlockSpec((_BLK, _D), lambda i: (i, 0)),
            pl.BlockSpec((1, _K), lambda i: (0, 0)),
            pl.BlockSpec((_K, _D), lambda i: (0, 0)),
        ],
        out_specs=[
            pl.BlockSpec((_BLK // 128, 128), lambda i: (i, 0)),
            pl.BlockSpec((1, 1), lambda i: (0, 0)),
        ],
        out_shape=[
            jax.ShapeDtypeStruct((_N // 128, 128), jnp.int32),
            jax.ShapeDtypeStruct((1, 1), jnp.float32),
        ],
    )(z, wsq, W)


def _sc_gather(table, idx3):
    mesh = plsc.VectorSubcoreMesh(core_axis_name="c", subcore_axis_name="s")

    @functools.partial(
        pl.kernel,
        mesh=mesh,
        out_type=jax.ShapeDtypeStruct((_N, _D), jnp.float32),
        scratch_types=[
            pltpu.VMEM((_BPW // _CHUNK, _CHUNK), jnp.int32),
            pltpu.VMEM((_BPW, _D), jnp.float32),
            pltpu.SemaphoreType.DMA,
        ],
        compiler_params=pltpu.CompilerParams(use_tc_tiling_on_sc=False),
    )
    def k(table_hbm, idx_hbm, out_hbm, idx_v, rows_v, sem):
        wid = lax.axis_index("s") * 2 + lax.axis_index("c")
        nrow = _BPW // _CHUNK
        pltpu.sync_copy(idx_hbm.at[pl.ds(wid * nrow, nrow)], idx_v)
        copies = []
        for j in range(nrow):
            copies.append(
                pltpu.async_copy(table_hbm.at[idx_v.at[j]],
                                 rows_v.at[pl.ds(j * _CHUNK, _CHUNK)], sem))
        for c in copies:
            c.wait()
        pltpu.sync_copy(rows_v, out_hbm.at[pl.ds(wid * _BPW, _BPW)])

    return k(table, idx3)


def kernel(z, W):
    wsq = jnp.sum(W ** 2, axis=1)[None, :]
    idxcc, loss2d = _tc_call(z, wsq, -2.0 * W)
    quantized = _sc_gather(W, idxcc)
    loss = loss2d[0, 0]
    return (quantized, loss, idxcc.reshape(_N, 1))
